# Initial kernel scaffold; baseline (speedup 1.0000x reference)
import jax
import jax.numpy as jnp
from jax.experimental import pallas as pl


def kernel(node_features, edge_features, memory, last_update, edge_times, neighbor_times, time_w, time_b, msg_w1, msg_b1, msg_w2, msg_b2, gru_wi, gru_wh, gru_bi, gru_bh, Wq, Wk, Wv, Wo, merge_w1, merge_b1, merge_w2, merge_b2, source_nodes, destination_nodes, negative_nodes, edge_idxs, neighbors, neighbor_edge_idxs):
    raise NotImplementedError("write your pallas kernel here")


# speedup vs baseline: 1.4564x; 1.4564x over previous
import functools

import jax
import jax.numpy as jnp
from jax import lax
from jax.experimental import pallas as pl
from jax.experimental.pallas import tpu as pltpu
from jax.experimental.pallas import tpu_sc as plsc

N = 10000
E = 320000
D = 128
B = 2000
K = 20

NC, NS = 2, 16
NW = NC * NS
NP = 10240
FIX = 4096
TBL = NP + FIX
HALF = 2048
NQ = 3 * B
NQP = 6144
KVROWS = K * NQP
RB = 512
RF = 256


def _widx():
    return lax.axis_index("s") * NC + lax.axis_index("c")


def _sc_gather_msg(memory, nf, ef, lu1d, idxn, idxe):
    mesh = plsc.VectorSubcoreMesh(core_axis_name="c", subcore_axis_name="s")

    @functools.partial(
        pl.kernel,
        out_type=[
            jax.ShapeDtypeStruct((FIX, D), jnp.float32),
            jax.ShapeDtypeStruct((FIX, D), jnp.float32),
            jax.ShapeDtypeStruct((HALF, D), jnp.float32),
            jax.ShapeDtypeStruct((FIX,), jnp.float32),
        ],
        mesh=mesh,
        scratch_types=[
            pltpu.VMEM((128,), jnp.int32),
            pltpu.VMEM((64,), jnp.int32),
            pltpu.VMEM((128, D), jnp.float32),
            pltpu.VMEM((64, D), jnp.float32),
            pltpu.VMEM((N,), jnp.float32),
            pltpu.VMEM((128,), jnp.float32),
            pltpu.SemaphoreType.DMA,
        ],
        compiler_params=pltpu.CompilerParams(needs_layout_passes=False),
    )
    def k(mem_h, nf_h, ef_h, lu_h, idxn_h, idxe_h,
          memcat_h, nfcat_h, efcat_h, lucat_h,
          idxn_v, idxe_v, rows_v, erows_v, lu_vm, lu_v, sem):
        wid = _widx()
        nb = wid * 128
        eb = wid * 64
        pltpu.sync_copy(lu_h, lu_vm)
        pltpu.sync_copy(idxn_h.at[wid], idxn_v)
        pltpu.async_copy(mem_h.at[idxn_v], rows_v, sem).wait()
        pltpu.sync_copy(rows_v, memcat_h.at[pl.ds(nb, 128)])
        pltpu.async_copy(nf_h.at[idxn_v], rows_v, sem).wait()
        pltpu.sync_copy(rows_v, nfcat_h.at[pl.ds(nb, 128)])
        for t in range(8):
            v = idxn_v[pl.ds(t * 16, 16)]
            lu_v[pl.ds(t * 16, 16)] = plsc.load_gather(lu_vm, [v])
        pltpu.sync_copy(lu_v, lucat_h.at[pl.ds(nb, 128)])
        pltpu.sync_copy(idxe_h.at[wid], idxe_v)
        pltpu.async_copy(ef_h.at[idxe_v], erows_v, sem).wait()
        pltpu.sync_copy(erows_v, efcat_h.at[pl.ds(eb, 64)])

    return k(memory, nf, ef, lu1d, idxn, idxe)


def _tc_tables_msgs(memory, nf, mem_cat, ef_cat, lu_cat, et2d, nf_cat,
                    tw, tb, w1, b1, w2p, b2p, wi_p, wh, gbi, gbh, wkv):
    nt = NP // RB
    nx = FIX // RB
    hb = HALF // RB

    def body(mem_r, nf_r, mself_r, mother_r, ef_r, lu_r, et_r, nfc_r,
             tw_r, tb_r, w1_r, b1_r, w2_r, b2_r, wi_r, wh_r, gbi_r, gbh_r,
             wkv_r, osum_r, okv_r):
        i = pl.program_id(0)

        @pl.when(i < nt)
        def _tables():
            s = mem_r[...] + nf_r[...]
            osum_r[...] = s
            okv_r[...] = jnp.dot(s, wkv_r[...],
                                 preferred_element_type=jnp.float32)

        @pl.when(i >= nt)
        def _msgs():
            mself = mself_r[...]
            mother = mother_r[...]
            efb = ef_r[...]
            dt = et_r[...] - lu_r[...]
            tenc = jnp.cos(dt * tw_r[...] + tb_r[...])
            raw = jnp.concatenate([mself, mother, efb, tenc], axis=1)
            h = jnp.maximum(
                jnp.dot(raw, w1_r[...], preferred_element_type=jnp.float32)
                + b1_r[...], 0.0)
            msg = jnp.dot(h, w2_r[...],
                          preferred_element_type=jnp.float32) + b2_r[...]
            gi = jnp.dot(msg, wi_r[...],
                         preferred_element_type=jnp.float32) + gbi_r[...]
            gh = jnp.dot(mself, wh_r[...],
                         preferred_element_type=jnp.float32) + gbh_r[...]
            r = jax.nn.sigmoid(gi[:, :D] + gh[:, :D])
            z = jax.nn.sigmoid(gi[:, D:2 * D] + gh[:, D:2 * D])
            n = jnp.tanh(gi[:, 2 * D:] + r * gh[:, 2 * D:])
            newm = (1.0 - z) * n + z * mself
            fsum = newm + nfc_r[...]
            osum_r[...] = fsum
            okv_r[...] = jnp.dot(fsum, wkv_r[...],
                                 preferred_element_type=jnp.float32)

    zmap = lambda i: (0, 0)
    tmap = lambda i: (jnp.minimum(i, nt - 1), 0)
    xmap = lambda i: (jnp.where(i < nt, 0, i - nt), 0)
    omap = lambda i: (jnp.where(i < nt, 0, (i - nt + hb) % nx), 0)
    emap = lambda i: (jnp.where(i < nt, 0, (i - nt) % hb), 0)

    return pl.pallas_call(
        body,
        grid=(nt + nx,),
        in_specs=[
            pl.BlockSpec((RB, D), tmap),
            pl.BlockSpec((RB, D), tmap),
            pl.BlockSpec((RB, D), xmap),
            pl.BlockSpec((RB, D), omap),
            pl.BlockSpec((RB, D), emap),
            pl.BlockSpec((RB, 1), xmap),
            pl.BlockSpec((RB, 1), emap),
            pl.BlockSpec((RB, D), xmap),
            pl.BlockSpec((1, D), zmap),
            pl.BlockSpec((1, D), zmap),
            pl.BlockSpec((4 * D, 2 * D), zmap),
            pl.BlockSpec((1, 2 * D), zmap),
            pl.BlockSpec((2 * D, D), zmap),
            pl.BlockSpec((1, D), zmap),
            pl.BlockSpec((D, 3 * D), zmap),
            pl.BlockSpec((D, 3 * D), zmap),
            pl.BlockSpec((1, 3 * D), zmap),
            pl.BlockSpec((1, 3 * D), zmap),
            pl.BlockSpec((D, 2 * D), zmap),
        ],
        out_specs=[
            pl.BlockSpec((RB, D), lambda i: (i, 0)),
            pl.BlockSpec((RB, 2 * D), lambda i: (i, 0)),
        ],
        out_shape=[
            jax.ShapeDtypeStruct((TBL, D), jnp.float32),
            jax.ShapeDtypeStruct((TBL, 2 * D), jnp.float32),
        ],
    )(memory, nf, mem_cat, mem_cat, ef_cat, lu_cat, et2d, nf_cat,
      tw, tb, w1, b1, w2p, b2p, wi_p, wh, gbi, gbh, wkv)


def _sc_gather_big(tkv_ext, nsum_ext, ef, aux_p, kvi, nei, sfi):
    mesh = plsc.VectorSubcoreMesh(core_axis_name="c", subcore_axis_name="s")
    ckv = KVROWS // NW // 128
    csf = NQP // NW // 96

    @functools.partial(
        pl.kernel,
        out_type=[
            jax.ShapeDtypeStruct((KVROWS, 2 * D), jnp.float32),
            jax.ShapeDtypeStruct((KVROWS, D), jnp.float32),
            jax.ShapeDtypeStruct((NQP, D), jnp.float32),
        ],
        mesh=mesh,
        scratch_types=[
            pltpu.VMEM((NP,), jnp.int32),
            pltpu.VMEM((128,), jnp.int32),
            pltpu.VMEM((96,), jnp.int32),
            pltpu.VMEM((128, 2 * D), jnp.float32),
            pltpu.VMEM((128, D), jnp.float32),
            pltpu.VMEM((96, D), jnp.float32),
            pltpu.SemaphoreType.DMA,
        ],
        compiler_params=pltpu.CompilerParams(needs_layout_passes=False),
    )
    def k(tkv_h, nsum_h, ef_h, aux_h, kvi_h, nei_h, sfi_h,
          gkv_h, gef_h, gsf_h,
          aux_v, idx_v, idxs_v, kvrows_v, efrows_v, sfrows_v, sem):
        wid = _widx()
        pltpu.sync_copy(aux_h, aux_v)

        def remap(ref, n):
            for t in range(n // 16):
                v = ref[pl.ds(t * 16, 16)]
                a = plsc.load_gather(aux_v, [v])
                ref[pl.ds(t * 16, 16)] = jnp.where(a >= 0, a + NP, v)

        def kv_body(j, _):
            base = wid * (ckv * 128) + j * 128
            pltpu.sync_copy(kvi_h.at[wid, j], idx_v)
            remap(idx_v, 128)
            pltpu.async_copy(tkv_h.at[idx_v], kvrows_v, sem).wait()
            pltpu.sync_copy(kvrows_v, gkv_h.at[pl.ds(base, 128)])
            pltpu.sync_copy(nei_h.at[wid, j], idx_v)
            pltpu.async_copy(ef_h.at[idx_v], efrows_v, sem).wait()
            pltpu.sync_copy(efrows_v, gef_h.at[pl.ds(base, 128)])
            return _

        lax.fori_loop(0, ckv, kv_body, 0)

        def sf_body(j, _):
            base = wid * (csf * 96) + j * 96
            pltpu.sync_copy(sfi_h.at[wid, j], idxs_v)
            remap(idxs_v, 96)
            pltpu.async_copy(nsum_h.at[idxs_v], sfrows_v, sem).wait()
            pltpu.sync_copy(sfrows_v, gsf_h.at[pl.ds(base, 96)])
            return _

        lax.fori_loop(0, csf, sf_body, 0)

    return k(tkv_ext, nsum_ext, ef, aux_p, kvi, nei, sfi)


def _tc_attn(g_kv, g_ef, g_sf, nt_p, ts_p, tw, tb, q0,
             wq1, w23, wo, mw1a, mw1b, mb1, mw2, mb2):
    nblk = NQP // RF
    dh = D // 2

    def body(kv_r, ef_r, sf_r, nt_r, ts_r, tw_r, tb_r, q0_r,
             wq1_r, w23_r, wo_r, mw1a_r, mw1b_r, mb1_r, mw2_r, mb2_r, out_r):
        sf = sf_r[...]
        q = jnp.dot(sf, wq1_r[...],
                    preferred_element_type=jnp.float32) + q0_r[...]
        ts = ts_r[...]
        twv = tw_r[...]
        tbv = tb_r[...]
        w23 = w23_r[...]
        l0s, l1s, vvs = [], [], []
        for kk_i in range(K):
            dt = ts - nt_r[:, kk_i:kk_i + 1]
            tenc = jnp.cos(dt * twv + tbv)
            et = jnp.concatenate([ef_r[kk_i], tenc], axis=1)
            kv = jnp.dot(et, w23,
                         preferred_element_type=jnp.float32) + kv_r[kk_i]
            kkp = q * kv[:, :D]
            l0s.append(jnp.sum(kkp[:, :dh], axis=1, keepdims=True))
            l1s.append(jnp.sum(kkp[:, dh:], axis=1, keepdims=True))
            vvs.append(kv[:, D:])
        scale = 1.0 / jnp.sqrt(float(dh))
        lg0 = jnp.concatenate(l0s, axis=1) * scale
        lg1 = jnp.concatenate(l1s, axis=1) * scale
        a0 = jnp.exp(lg0 - jnp.max(lg0, axis=1, keepdims=True))
        a0 = a0 / jnp.sum(a0, axis=1, keepdims=True)
        a1 = jnp.exp(lg1 - jnp.max(lg1, axis=1, keepdims=True))
        a1 = a1 / jnp.sum(a1, axis=1, keepdims=True)
        o0 = a0[:, 0:1] * vvs[0][:, :dh]
        o1 = a1[:, 0:1] * vvs[0][:, dh:]
        for kk_i in range(1, K):
            o0 = o0 + a0[:, kk_i:kk_i + 1] * vvs[kk_i][:, :dh]
            o1 = o1 + a1[:, kk_i:kk_i + 1] * vvs[kk_i][:, dh:]
        attn = jnp.concatenate([o0, o1], axis=1)
        attn = jnp.dot(attn, wo_r[...], preferred_element_type=jnp.float32)
        h1 = jnp.maximum(
            jnp.dot(attn, mw1a_r[...], preferred_element_type=jnp.float32)
            + jnp.dot(sf, mw1b_r[...], preferred_element_type=jnp.float32)
            + mb1_r[...], 0.0)
        out_r[...] = jnp.dot(h1, mw2_r[...],
                             preferred_element_type=jnp.float32) + mb2_r[...]

    zmap = lambda i: (0, 0)

    return pl.pallas_call(
        body,
        grid=(nblk,),
        in_specs=[
            pl.BlockSpec((K, RF, 2 * D), lambda i: (0, i, 0)),
            pl.BlockSpec((K, RF, D), lambda i: (0, i, 0)),
            pl.BlockSpec((RF, D), lambda i: (i, 0)),
            pl.BlockSpec((RF, K), lambda i: (i, 0)),
            pl.BlockSpec((RF, 1), lambda i: (i, 0)),
            pl.BlockSpec((1, D), zmap),
            pl.BlockSpec((1, D), zmap),
            pl.BlockSpec((1, D), zmap),
            pl.BlockSpec((D, D), zmap),
            pl.BlockSpec((2 * D, 2 * D), zmap),
            pl.BlockSpec((D, D), zmap),
            pl.BlockSpec((D, D), zmap),
            pl.BlockSpec((D, D), zmap),
            pl.BlockSpec((1, D), zmap),
            pl.BlockSpec((D, D), zmap),
            pl.BlockSpec((1, D), zmap),
        ],
        out_specs=pl.BlockSpec((RF, D), lambda i: (i, 0)),
        out_shape=jax.ShapeDtypeStruct((NQP, D), jnp.float32),
    )(g_kv, g_ef, g_sf, nt_p, ts_p, tw, tb, q0,
      wq1, w23, wo, mw1a, mw1b, mb1, mw2, mb2)


def kernel(node_features, edge_features, memory, last_update, edge_times,
           neighbor_times, time_w, time_b, msg_w1, msg_b1, msg_w2, msg_b2,
           gru_wi, gru_wh, gru_bi, gru_bh, Wq, Wk, Wv, Wo,
           merge_w1, merge_b1, merge_w2, merge_b2,
           source_nodes, destination_nodes, negative_nodes,
           edge_idxs, neighbors, neighbor_edge_idxs):
    i32 = jnp.int32
    pad48 = jnp.zeros((HALF - B,), i32)
    nodes_pad = jnp.concatenate(
        [source_nodes, pad48, destination_nodes, pad48])
    idxn = nodes_pad.reshape(NW, 128)
    idxe = jnp.concatenate([edge_idxs, pad48]).reshape(NW, 64)

    rows = jnp.arange(FIX, dtype=i32)
    real = (rows < B) | ((rows >= HALF) & (rows < HALF + B))
    ids = jnp.where(real, rows, -1)
    aux = jnp.full((N,), -1, i32).at[nodes_pad].max(ids)
    aux_p = jnp.concatenate([aux, jnp.full((NP - N,), -1, i32)])

    et2d = jnp.concatenate(
        [edge_times, jnp.zeros((HALF - B,))]).reshape(HALF, 1)

    kvi = jnp.pad(neighbors.T.astype(i32),
                  ((0, 0), (0, NQP - NQ))).reshape(NW, -1, 128)
    nei = jnp.pad(neighbor_edge_idxs.T.astype(i32),
                  ((0, 0), (0, NQP - NQ))).reshape(NW, -1, 128)
    nodes_all = jnp.concatenate(
        [source_nodes, destination_nodes, negative_nodes])
    sfi = jnp.pad(nodes_all, (0, NQP - NQ)).reshape(NW, -1, 96)
    nt_p = jnp.pad(neighbor_times, ((0, NQP - NQ), (0, 0)))
    ts_p = jnp.pad(jnp.concatenate([edge_times] * 3),
                   (0, NQP - NQ)).reshape(NQP, 1)

    tw = time_w.reshape(1, D)
    tb = time_b.reshape(1, D)
    b1 = msg_b1.reshape(1, 2 * D)
    w2p = jnp.pad(msg_w2, ((0, 0), (0, D - msg_w2.shape[1])))
    b2p = jnp.pad(msg_b2, (0, D - msg_b2.shape[0])).reshape(1, D)
    wi_p = jnp.pad(gru_wi, ((0, D - gru_wi.shape[0]), (0, 0)))
    gbi = gru_bi.reshape(1, 3 * D)
    gbh = gru_bh.reshape(1, 3 * D)
    wkv = jnp.concatenate([Wk[:D], Wv[:D]], axis=1)
    w23 = jnp.concatenate(
        [jnp.concatenate([Wk[D:2 * D], Wv[D:2 * D]], axis=1),
         jnp.concatenate([Wk[2 * D:], Wv[2 * D:]], axis=1)], axis=0)
    wq1 = Wq[:D]
    q0 = (jnp.cos(time_b) @ Wq[D:]).reshape(1, D)
    mw1a = merge_w1[:D]
    mw1b = merge_w1[D:]
    mb1 = merge_b1.reshape(1, D)
    mb2 = merge_b2.reshape(1, D)

    mem_cat, nf_cat, ef_cat, lu_cat = _sc_gather_msg(
        memory, node_features, edge_features, last_update, idxn, idxe)

    nsum_ext, tkv_ext = _tc_tables_msgs(
        memory, node_features, mem_cat, ef_cat, lu_cat.reshape(FIX, 1),
        et2d, nf_cat,
        tw, tb, msg_w1, b1, w2p, b2p, wi_p, gru_wh, gbi, gbh, wkv)

    g_kv, g_ef, g_sf = _sc_gather_big(
        tkv_ext, nsum_ext, edge_features, aux_p, kvi, nei, sfi)

    emb_p = _tc_attn(
        g_kv.reshape(K, NQP, 2 * D), g_ef.reshape(K, NQP, D), g_sf,
        nt_p, ts_p, tw, tb, q0, wq1, w23, Wo, mw1a, mw1b, mb1,
        merge_w2, mb2)

    return emb_p[:NQ]
